# Initial kernel scaffold; baseline (speedup 1.0000x reference)
import jax
import jax.numpy as jnp
from jax.experimental import pallas as pl


def kernel(input, connections, weights):
    raise NotImplementedError("write your pallas kernel here")


# speedup vs baseline: 3.3580x; 3.3580x over previous
import functools

import jax
import jax.numpy as jnp
from jax import lax
from jax.experimental import pallas as pl
from jax.experimental.pallas import tpu as pltpu
from jax.experimental.pallas import tpu_sc as plsc

BATCH = 2048
IN_FEATURES = 2048
OUT_FEATURES = 2048
WPO = 16

_NC = 2
_NS = 16
_NW = _NC * _NS
_O_PER_W = OUT_FEATURES // _NW
_CHUNK_ROWS = 16
_N_CHUNKS = _O_PER_W // _CHUNK_ROWS
_CHUNK_WORDS = _CHUNK_ROWS * IN_FEATURES


_TAPS_PER_W = WPO * _O_PER_W


def _densify_sc(conn_f, wt_f):
    mesh = plsc.VectorSubcoreMesh(core_axis_name="c", subcore_axis_name="s")

    @functools.partial(
        pl.kernel,
        mesh=mesh,
        out_type=jax.ShapeDtypeStruct((OUT_FEATURES * IN_FEATURES,), jnp.float32),
        scratch_types=[
            pltpu.VMEM((_TAPS_PER_W,), jnp.int32),
            pltpu.VMEM((_TAPS_PER_W,), jnp.float32),
            pltpu.VMEM((_CHUNK_WORDS,), jnp.float32),
        ],
        compiler_params=pltpu.CompilerParams(needs_layout_passes=False),
    )
    def densify(conn_hbm, wt_hbm, wd_hbm, conn_v, wt_v, buf):
        wid = lax.axis_index("s") * _NC + lax.axis_index("c")
        base = wid * _O_PER_W
        pltpu.sync_copy(conn_hbm.at[pl.ds(wid * _TAPS_PER_W, _TAPS_PER_W)], conn_v)
        pltpu.sync_copy(wt_hbm.at[pl.ds(wid * _TAPS_PER_W, _TAPS_PER_W)], wt_v)
        row_off = lax.iota(jnp.int32, 16) * IN_FEATURES
        zeros16 = jnp.zeros((16,), jnp.float32)
        for c in range(_N_CHUNKS):
            def _zero(j, _):
                buf[pl.ds(j * 16, 16)] = zeros16
                return 0
            lax.fori_loop(0, _CHUNK_WORDS // 16, _zero, 0)
            for w in range(WPO):
                cols = conn_v[pl.ds(w * _O_PER_W + c * _CHUNK_ROWS, 16)]
                vals = wt_v[pl.ds(w * _O_PER_W + c * _CHUNK_ROWS, 16)]
                plsc.addupdate_scatter(buf, [row_off + cols], vals)
            dst = (base + c * _CHUNK_ROWS) * IN_FEATURES
            pltpu.sync_copy(buf, wd_hbm.at[pl.ds(dst, _CHUNK_WORDS)])

    return densify(conn_f, wt_f)


def _matmul_tc(x, wd):
    bb, bo = 512, 512

    def mm(x_ref, w_ref, o_ref):
        xb = x_ref[...].astype(jnp.bfloat16)
        wb = w_ref[...].astype(jnp.bfloat16)
        o_ref[...] = lax.dot_general(
            xb, wb, (((1,), (1,)), ((), ())),
            preferred_element_type=jnp.float32)

    return pl.pallas_call(
        mm,
        grid=(BATCH // bb, OUT_FEATURES // bo),
        in_specs=[
            pl.BlockSpec((bb, IN_FEATURES), lambda b, o: (b, 0)),
            pl.BlockSpec((bo, IN_FEATURES), lambda b, o: (o, 0)),
        ],
        out_specs=pl.BlockSpec((bb, bo), lambda b, o: (b, o)),
        out_shape=jax.ShapeDtypeStruct((BATCH, OUT_FEATURES), jnp.float32),
    )(x, wd)


def kernel(input, connections, weights):
    conn_f = connections.reshape(_NW, _O_PER_W, WPO).transpose(0, 2, 1).reshape(-1)
    wt_f = weights.reshape(_NW, _O_PER_W, WPO).transpose(0, 2, 1).reshape(-1)
    wd = _densify_sc(conn_f, wt_f).reshape(OUT_FEATURES, IN_FEATURES)
    return _matmul_tc(input, wd)


# speedup vs baseline: 3.8526x; 1.1473x over previous
import functools

import jax
import jax.numpy as jnp
from jax import lax
from jax.experimental import pallas as pl
from jax.experimental.pallas import tpu as pltpu
from jax.experimental.pallas import tpu_sc as plsc

BATCH = 2048
IN_FEATURES = 2048
OUT_FEATURES = 2048
WPO = 16

_NC = 2
_NS = 16
_NW = _NC * _NS
_O_PER_W = OUT_FEATURES // _NW
_CHUNK_ROWS = 16
_N_CHUNKS = _O_PER_W // _CHUNK_ROWS
_CHUNK_WORDS = _CHUNK_ROWS * IN_FEATURES


_TAPS_PER_W = WPO * _O_PER_W


def _densify_sc(conn_f, wt_f):
    mesh = plsc.VectorSubcoreMesh(core_axis_name="c", subcore_axis_name="s")

    @functools.partial(
        pl.kernel,
        mesh=mesh,
        out_type=jax.ShapeDtypeStruct((OUT_FEATURES * IN_FEATURES,), jnp.float32),
        scratch_types=[
            pltpu.VMEM((_TAPS_PER_W,), jnp.int32),
            pltpu.VMEM((_TAPS_PER_W,), jnp.float32),
            pltpu.VMEM((_CHUNK_WORDS,), jnp.float32),
            pltpu.VMEM((_CHUNK_WORDS,), jnp.float32),
            pltpu.SemaphoreType.DMA,
            pltpu.SemaphoreType.DMA,
        ],
        compiler_params=pltpu.CompilerParams(needs_layout_passes=False),
    )
    def densify(conn_hbm, wt_hbm, wd_hbm, conn_v, wt_v, buf0, buf1, sem0, sem1):
        wid = lax.axis_index("s") * _NC + lax.axis_index("c")
        base = wid * _O_PER_W
        pltpu.sync_copy(conn_hbm.at[pl.ds(wid * _TAPS_PER_W, _TAPS_PER_W)], conn_v)
        pltpu.sync_copy(wt_hbm.at[pl.ds(wid * _TAPS_PER_W, _TAPS_PER_W)], wt_v)
        row_off = lax.iota(jnp.int32, 16) * IN_FEATURES
        zeros16 = jnp.zeros((16,), jnp.float32)
        bufs = (buf0, buf1)
        sems = (sem0, sem1)
        copies = [None, None]
        for c in range(_N_CHUNKS):
            slot = c % 2
            buf = bufs[slot]
            if copies[slot] is not None:
                copies[slot].wait()
            def _zero(j, _):
                buf[pl.ds(j * 16, 16)] = zeros16
                return 0
            lax.fori_loop(0, _CHUNK_WORDS // 16, _zero, 0)
            for w in range(WPO):
                cols = conn_v[pl.ds(w * _O_PER_W + c * _CHUNK_ROWS, 16)]
                vals = wt_v[pl.ds(w * _O_PER_W + c * _CHUNK_ROWS, 16)]
                plsc.addupdate_scatter(buf, [row_off + cols], vals)
            dst = (base + c * _CHUNK_ROWS) * IN_FEATURES
            copies[slot] = pltpu.async_copy(
                buf, wd_hbm.at[pl.ds(dst, _CHUNK_WORDS)], sems[slot])
        for cp in copies:
            if cp is not None:
                cp.wait()

    return densify(conn_f, wt_f)


def _matmul_tc(x, wd):
    bo = 512

    def mm(x_ref, w_ref, o_ref):
        xb = x_ref[...].astype(jnp.bfloat16)
        wb = w_ref[...].astype(jnp.bfloat16)
        o_ref[...] = lax.dot_general(
            xb, wb, (((1,), (1,)), ((), ())),
            preferred_element_type=jnp.float32)

    return pl.pallas_call(
        mm,
        grid=(OUT_FEATURES // bo,),
        in_specs=[
            pl.BlockSpec((BATCH, IN_FEATURES), lambda o: (0, 0)),
            pl.BlockSpec((bo, IN_FEATURES), lambda o: (o, 0)),
        ],
        out_specs=pl.BlockSpec((BATCH, bo), lambda o: (0, o)),
        out_shape=jax.ShapeDtypeStruct((BATCH, OUT_FEATURES), jnp.float32),
    )(x, wd)


def kernel(input, connections, weights):
    conn_f = connections.reshape(_NW, _O_PER_W, WPO).transpose(0, 2, 1).reshape(-1)
    wt_f = weights.reshape(_NW, _O_PER_W, WPO).transpose(0, 2, 1).reshape(-1)
    wd = _densify_sc(conn_f, wt_f).reshape(OUT_FEATURES, IN_FEATURES)
    return _matmul_tc(input, wd)


# speedup vs baseline: 6.8752x; 1.7846x over previous
import functools

import jax
import jax.numpy as jnp
from jax import lax
from jax.experimental import pallas as pl
from jax.experimental.pallas import tpu as pltpu
from jax.experimental.pallas import tpu_sc as plsc

BATCH = 2048
IN_FEATURES = 2048
OUT_FEATURES = 2048
WPO = 16

_NC = 2
_NS = 16
_NW = _NC * _NS
_O_PER_W = OUT_FEATURES // _NW
_CHUNK_ROWS = 16
_N_CHUNKS = _O_PER_W // _CHUNK_ROWS


def _densify_sc(connections, weights):
    mesh = plsc.VectorSubcoreMesh(core_axis_name="c", subcore_axis_name="s")

    @functools.partial(
        pl.kernel,
        mesh=mesh,
        out_type=jax.ShapeDtypeStruct((OUT_FEATURES, IN_FEATURES), jnp.float32),
        scratch_types=[
            pltpu.VMEM((_O_PER_W, WPO), jnp.int32),
            pltpu.VMEM((_O_PER_W, WPO), jnp.float32),
            pltpu.VMEM((_CHUNK_ROWS, IN_FEATURES), jnp.float32),
            pltpu.VMEM((_CHUNK_ROWS, IN_FEATURES), jnp.float32),
            pltpu.SemaphoreType.DMA,
            pltpu.SemaphoreType.DMA,
        ],
        compiler_params=pltpu.CompilerParams(needs_layout_passes=False),
    )
    def densify(conn_hbm, wt_hbm, wd_hbm, conn_v, wt_v, buf0, buf1, sem0, sem1):
        wid = lax.axis_index("s") * _NC + lax.axis_index("c")
        base = wid * _O_PER_W
        pltpu.sync_copy(conn_hbm.at[pl.ds(base, _O_PER_W), :], conn_v)
        pltpu.sync_copy(wt_hbm.at[pl.ds(base, _O_PER_W), :], wt_v)
        lane = lax.iota(jnp.int32, 16)
        zeros16 = jnp.zeros((16,), jnp.float32)
        bufs = (buf0, buf1)
        sems = (sem0, sem1)

        for buf in bufs:
            def _zero(j, _):
                for r in range(_CHUNK_ROWS):
                    buf[r, pl.ds(j * 16, 16)] = zeros16
                return 0
            lax.fori_loop(0, IN_FEATURES // 16, _zero, 0)

        def taps(c, w):
            ridx = c * _CHUNK_ROWS + lane
            widx = jnp.full((16,), w, jnp.int32)
            cols = plsc.load_gather(conn_v, [ridx, widx])
            vals = plsc.load_gather(wt_v, [ridx, widx])
            return cols, vals

        copies = [None, None]
        for c in range(_N_CHUNKS):
            slot = c % 2
            buf = bufs[slot]
            if copies[slot] is not None:
                copies[slot].wait()
                for w in range(WPO):
                    cols_prev, _ = taps(c - 2, w)
                    plsc.store_scatter(buf, [lane, cols_prev], zeros16)
            for w in range(WPO):
                cols, vals = taps(c, w)
                plsc.addupdate_scatter(buf, [lane, cols], vals)
            row0 = base + c * _CHUNK_ROWS
            copies[slot] = pltpu.async_copy(
                buf, wd_hbm.at[pl.ds(row0, _CHUNK_ROWS), :], sems[slot])
        for cp in copies:
            if cp is not None:
                cp.wait()

    return densify(connections, weights)


def _matmul_tc(x, wd):
    bo = 512

    def mm(x_ref, w_ref, o_ref):
        xb = x_ref[...].astype(jnp.bfloat16)
        wb = w_ref[...].astype(jnp.bfloat16)
        o_ref[...] = lax.dot_general(
            xb, wb, (((1,), (1,)), ((), ())),
            preferred_element_type=jnp.float32)

    return pl.pallas_call(
        mm,
        grid=(OUT_FEATURES // bo,),
        in_specs=[
            pl.BlockSpec((BATCH, IN_FEATURES), lambda o: (0, 0)),
            pl.BlockSpec((bo, IN_FEATURES), lambda o: (o, 0)),
        ],
        out_specs=pl.BlockSpec((BATCH, bo), lambda o: (0, o)),
        out_shape=jax.ShapeDtypeStruct((BATCH, OUT_FEATURES), jnp.float32),
    )(x, wd)


def kernel(input, connections, weights):
    wd = _densify_sc(connections, weights)
    return _matmul_tc(input, wd)
